# baseline (speedup 1.0000x reference)
import numpy as np
import jax
import jax.numpy as jnp
from jax import lax
from jax.experimental import pallas as pl
from jax.experimental.pallas import tpu as pltpu
from jax.experimental.pallas import tpu_sc as plsc

N_G = 10000
N_P = 1000
D = 128
OUT = 16
NC = 2
NS = 16
CHUNK = 128
NB = 8
GH = 5008
PH = 512
N_G_OUT = 2 * GH
N_P_OUT = 2 * PH
CG = 160
CP = 64

_PIDX = np.concatenate(
    [np.concatenate([q * 32 + 2 * np.arange(16), q * 32 + 2 * np.arange(16) + 1])
     for q in range(4)]).astype(np.int32)


def _seg_body(msg_gg, msg_gp, ggsp, ggh, ggdl, ggwm, gpsp, gph, gpdl, gpwm,
              tg_out, tp_out,
              msg_sp, acc_g, acc_p, sv, hv, dlv, wv, pk, rf, sem):
  c = lax.axis_index("c")
  s = lax.axis_index("s")

  def zrow(i, _):
    for d8 in range(8):
      rf[i, pl.ds(d8 * 16, 16)] = jnp.zeros((16,), jnp.float32)
    return 0
  lax.fori_loop(0, CHUNK, zrow, 0)

  base_g = s * 312
  for i in range(2):
    pltpu.sync_copy(rf, acc_g.at[pl.ds(base_g + i * 128, 128)])
  pltpu.sync_copy(rf.at[pl.ds(0, 56)], acc_g.at[pl.ds(base_g + 256, 56)])
  pltpu.sync_copy(rf.at[pl.ds(0, 32)], acc_p.at[pl.ds(s * 32, 32)])
  @pl.when(s == 0)
  def _():
    pltpu.sync_copy(rf.at[pl.ds(0, 16)], acc_g.at[pl.ds(4992, 16)])

  base_m = s * 320
  pltpu.sync_copy(msg_gg.at[pl.ds(base_m, 320)], msg_sp.at[pl.ds(base_m, 320)])
  plsc.subcore_barrier()

  def do_rel(n_blocks, sps, hs, dls, wms, acc):
    def block_body(bb, _):
      off = pl.multiple_of(bb * NB, NB)
      pltpu.sync_copy(sps.at[s, pl.ds(off, NB)], sv)
      pltpu.sync_copy(hs.at[s, pl.ds(off, NB)], hv)
      pltpu.sync_copy(dls.at[c, s, pl.ds(off, NB)], dlv)
      pltpu.sync_copy(wms.at[c, s, pl.ds(off, NB)], wv)

      def chunk_body(jj, _):
        pltpu.async_copy(msg_sp.at[sv.at[jj]], pk, sem).wait()

        for g in range(CHUNK // 16):
          w16 = wv[jj, pl.ds(g * 16, 16)]
          h16 = hv[jj, pl.ds(g * 16, 16)]
          for lane in range(16):
            k = g * 16 + lane
            w_s = w16[lane]
            h_s = h16[lane]
            for q in range(4):
              lo = pk[k, pl.ds(q * 16, 16)]
              hi = pk[k, pl.ds(64 + q * 16, 16)]
              w32 = jnp.where(h_s == 0, lo, hi)
              a = lax.bitcast_convert_type(
                  lax.shift_left(w32, 16), jnp.float32)
              b = lax.bitcast_convert_type(
                  lax.bitwise_and(w32, jnp.int32(-65536)), jnp.float32)
              rf[k, pl.ds(q * 32, 16)] = a * w_s
              rf[k, pl.ds(q * 32 + 16, 16)] = b * w_s
        pltpu.sync_copy(rf, acc.at[dlv.at[jj]], add=True)
        return 0
      lax.fori_loop(0, NB, chunk_body, 0)
      return 0
    lax.fori_loop(0, n_blocks, block_body, 0)

  do_rel(CG // NB, ggsp, ggh, ggdl, ggwm, acc_g)
  plsc.subcore_barrier()

  pltpu.sync_copy(msg_gp.at[pl.ds(base_m, 320)], msg_sp.at[pl.ds(base_m, 320)])
  plsc.subcore_barrier()

  do_rel(CP // NB, gpsp, gph, gpdl, gpwm, acc_p)
  plsc.subcore_barrier()

  go = c * GH + base_g
  pltpu.sync_copy(acc_g.at[pl.ds(base_g, 312)], tg_out.at[pl.ds(go, 312)])
  pltpu.sync_copy(acc_p.at[pl.ds(s * 32, 32)],
                  tp_out.at[pl.ds(c * PH + s * 32, 32)])
  @pl.when(s == 0)
  def _():
    pltpu.sync_copy(acc_g.at[pl.ds(4992, 16)],
                    tg_out.at[pl.ds(c * GH + 4992, 16)])


def _seg_kernel():
  return pl.kernel(
    _seg_body,
    out_type=(
        jax.ShapeDtypeStruct((N_G_OUT, D), jnp.float32),
        jax.ShapeDtypeStruct((N_P_OUT, D), jnp.float32),
    ),
    mesh=plsc.VectorSubcoreMesh(
        core_axis_name="c", subcore_axis_name="s", num_cores=NC,
        num_subcores=NS),
    scratch_types=(
        pltpu.VMEM_SHARED((5120, D), jnp.int32),
        pltpu.VMEM_SHARED((GH, D), jnp.float32),
        pltpu.VMEM_SHARED((PH, D), jnp.float32),
        pltpu.VMEM((NB, CHUNK), jnp.int32),
        pltpu.VMEM((NB, CHUNK), jnp.int32),
        pltpu.VMEM((NB, CHUNK), jnp.int32),
        pltpu.VMEM((NB, CHUNK), jnp.float32),
        pltpu.VMEM((CHUNK, D), jnp.int32),
        pltpu.VMEM((CHUNK, D), jnp.float32),
        pltpu.SemaphoreType.DMA,
    ),
  )


def _mm2_body(x_ref, w1_ref, b1_ref, w2_ref, b2_ref, o1_ref, o2_ref):
  x = x_ref[...]
  o1_ref[...] = jnp.dot(x, w1_ref[...],
                        preferred_element_type=jnp.float32) + b1_ref[...]
  o2_ref[...] = jnp.dot(x, w2_ref[...],
                        preferred_element_type=jnp.float32) + b2_ref[...]


def _mm2(h, w1, b1, w2, b2):
  blk = 2000
  grid = N_G // blk
  return pl.pallas_call(
      _mm2_body,
      grid=(grid,),
      in_specs=[
          pl.BlockSpec((blk, D), lambda i: (i, 0)),
          pl.BlockSpec((D, D), lambda i: (0, 0)),
          pl.BlockSpec((1, D), lambda i: (0, 0)),
          pl.BlockSpec((D, D), lambda i: (0, 0)),
          pl.BlockSpec((1, D), lambda i: (0, 0)),
      ],
      out_specs=[
          pl.BlockSpec((blk, D), lambda i: (i, 0)),
          pl.BlockSpec((blk, D), lambda i: (i, 0)),
      ],
      out_shape=[
          jax.ShapeDtypeStruct((N_G, D), jnp.float32),
          jax.ShapeDtypeStruct((N_G, D), jnp.float32),
      ],
  )(h, w1, b1.reshape(1, D), w2, b2.reshape(1, D))


def _finish_math(t, h, wa, ba, g, b, sk):
  z = jnp.dot(t, wa, preferred_element_type=jnp.float32) + ba
  a = jax.nn.sigmoid(sk)
  u = z * a + h * (1.0 - a)
  m = jnp.mean(u, axis=-1, keepdims=True)
  v = jnp.mean((u - m) ** 2, axis=-1, keepdims=True)
  return (u - m) * lax.rsqrt(v + 1e-5) * g + b


def _fin_body(sk_ref, t_ref, h_ref, wa_ref, ba_ref, g_ref, b_ref, o_ref):
  o_ref[...] = _finish_math(t_ref[...], h_ref[...], wa_ref[...], ba_ref[...],
                            g_ref[...], b_ref[...], sk_ref[0])


def _fin(t, h, wa, ba, g, b, sk, n, blk):
  grid = n // blk
  return pl.pallas_call(
      _fin_body,
      grid=(grid,),
      in_specs=[
          pl.BlockSpec(memory_space=pltpu.SMEM),
          pl.BlockSpec((blk, D), lambda i: (i, 0)),
          pl.BlockSpec((blk, D), lambda i: (i, 0)),
          pl.BlockSpec((D, D), lambda i: (0, 0)),
          pl.BlockSpec((1, D), lambda i: (0, 0)),
          pl.BlockSpec((1, D), lambda i: (0, 0)),
          pl.BlockSpec((1, D), lambda i: (0, 0)),
      ],
      out_specs=pl.BlockSpec((blk, D), lambda i: (i, 0)),
      out_shape=jax.ShapeDtypeStruct((n, D), jnp.float32),
  )(jnp.reshape(sk, (1,)), t, h, wa, ba.reshape(1, D), g.reshape(1, D),
    b.reshape(1, D))


def _fin_head_body(sk_ref, t_ref, h_ref, wa_ref, ba_ref, g_ref, b_ref,
                   wo_ref, bo_ref, o_ref):
  hp = _finish_math(t_ref[...], h_ref[...], wa_ref[...], ba_ref[...],
                    g_ref[...], b_ref[...], sk_ref[0])
  o_ref[...] = jnp.dot(hp, wo_ref[...],
                       preferred_element_type=jnp.float32) + bo_ref[...]


def _fin_head(t, h, wa, ba, g, b, sk, wo, bo):
  return pl.pallas_call(
      _fin_head_body,
      grid=(1,),
      in_specs=[
          pl.BlockSpec(memory_space=pltpu.SMEM),
          pl.BlockSpec((N_P, D), lambda i: (0, 0)),
          pl.BlockSpec((N_P, D), lambda i: (0, 0)),
          pl.BlockSpec((D, D), lambda i: (0, 0)),
          pl.BlockSpec((1, D), lambda i: (0, 0)),
          pl.BlockSpec((1, D), lambda i: (0, 0)),
          pl.BlockSpec((1, D), lambda i: (0, 0)),
          pl.BlockSpec((D, OUT), lambda i: (0, 0)),
          pl.BlockSpec((1, OUT), lambda i: (0, 0)),
      ],
      out_specs=pl.BlockSpec((N_P, OUT), lambda i: (0, 0)),
      out_shape=jax.ShapeDtypeStruct((N_P, OUT), jnp.float32),
  )(jnp.reshape(sk, (1,)), t, h, wa, ba.reshape(1, D), g.reshape(1, D),
    b.reshape(1, D), wo, bo.reshape(1, OUT))


def _pack_msg(msg):
  bf = jnp.pad(msg.astype(jnp.bfloat16), ((0, 240), (0, 0)))
  return lax.bitcast_convert_type(bf.reshape(5120, D, 2), jnp.int32)


def _prep_edges(src, dst, w, n_chunks, half, pad_dst):
  e = src.shape[0]
  tot = NS * n_chunks * CHUNK
  pad = tot - e
  src = jnp.pad(src, (0, pad))
  dst = jnp.concatenate(
      [dst, jnp.full((pad,), pad_dst, jnp.int32)])
  w = jnp.pad(w, (0, pad))
  sp = (src >> 1).reshape(NS, n_chunks, CHUNK)
  h = (src & 1).reshape(NS, n_chunks, CHUNK)
  spread = (jnp.arange(tot, dtype=jnp.int32) % half)
  dls, wms = [], []
  for cc in range(NC):
    inb = (dst >= cc * half) & (dst < (cc + 1) * half)
    dls.append(jnp.where(inb, dst - cc * half, spread))
    wms.append(jnp.where(inb, w, 0.0))
  dl = jnp.stack(dls).reshape(NC, NS, n_chunks, CHUNK)
  wm = jnp.stack(wms).reshape(NC, NS, n_chunks, CHUNK)
  return sp, h, dl, wm


def kernel(nid_gene, nid_patient, gg_src, gg_dst, gp_src, gp_dst, w_gg, w_gp,
           emb_gene, emb_patient, Wr_gg, br_gg, Wr_gp, br_gp,
           Wa_g, ba_g, Wa_p, ba_p, ln_g_w, ln_g_b, ln_p_w, ln_p_b,
           skip, W_out, b_out):
  h_g = jnp.take(emb_gene, nid_gene, axis=0)
  h_p = jnp.take(emb_patient, nid_patient, axis=0)

  ggsp, ggh, ggdl, ggwm = _prep_edges(gg_src, gg_dst, w_gg, CG, GH, 0)
  gpsp, gph, gpdl, gpwm = _prep_edges(gp_src, gp_dst, w_gp, CP, PH, N_P)

  pidx = jnp.asarray(_PIDX)
  seg = _seg_kernel()

  for l in range(2):
    msg_gg, msg_gp = _mm2(h_g, Wr_gg[l], br_gg[l], Wr_gp[l], br_gp[l])
    tg, tp = seg(_pack_msg(msg_gg), _pack_msg(msg_gp),
                 ggsp, ggh, ggdl, ggwm, gpsp, gph, gpdl, gpwm)
    tg = tg[:N_G]
    tp = tp[:N_P]
    wa_g = Wa_g[l][pidx]
    wa_p = Wa_p[l][pidx]
    if l == 0:
      h_g = _fin(tg, h_g, wa_g, ba_g[l], ln_g_w[l], ln_g_b[l],
                 skip[l, 0], N_G, 2000)
      h_p = _fin(tp, h_p, wa_p, ba_p[l], ln_p_w[l], ln_p_b[l],
                 skip[l, 1], N_P, N_P)
    else:
      logits = _fin_head(tp, h_p, wa_p, ba_p[l], ln_p_w[l],
                         ln_p_b[l], skip[l, 1], W_out, b_out)
  return logits


# speedup vs baseline: 3.8594x; 3.8594x over previous
import numpy as np
import jax
import jax.numpy as jnp
from jax import lax
from jax.experimental import pallas as pl
from jax.experimental.pallas import tpu as pltpu
from jax.experimental.pallas import tpu_sc as plsc

N_G = 10000
N_G_PAD = 10048
N_P = 1000
N_P_PAD = 1024
D = 128
OUT = 16
NC = 2
NS = 16
CHUNK = 128
NB = 8
CG = 80
NBP = 5
CPB = 5


def _seg_body(msg_gg, msg_gp, ggs, ggd, ggw, gps, gpd, gpw,
              tg_out, tp_out,
              acc_g, acc_p, sv, dv, wv, rows0, rows1, sem0, sem1):
  c = lax.axis_index("c")
  s = lax.axis_index("s")
  t = c * NS + s
  rows_bufs = (rows0, rows1)
  sems = (sem0, sem1)
  rf = rows0

  def zrow(i, _):
    for d8 in range(8):
      rf[i, pl.ds(d8 * 16, 16)] = jnp.zeros((16,), jnp.float32)
    return 0
  lax.fori_loop(0, CHUNK, zrow, 0)

  base_g = s * 624
  for i in range(4):
    pltpu.sync_copy(rf, acc_g.at[pl.ds(base_g + i * 128, 128)])
  pltpu.sync_copy(rf.at[pl.ds(0, 112)], acc_g.at[pl.ds(base_g + 512, 112)])
  pltpu.sync_copy(rf.at[pl.ds(0, 64)], acc_p.at[pl.ds(s * 64, 64)])
  @pl.when(s == 0)
  def _():
    pltpu.sync_copy(rf.at[pl.ds(0, 64)], acc_g.at[pl.ds(9984, 64)])
  plsc.subcore_barrier()

  def do_rel(n_blocks, nb, stage, msg, acc):
    def block_body(bb, _):
      stage(bb)
      pltpu.async_copy(msg.at[sv.at[0]], rows_bufs[0], sems[0])
      for jj in range(nb):
        cur, csem = rows_bufs[jj % 2], sems[jj % 2]
        pltpu.make_async_copy(msg.at[sv.at[jj]], cur, csem).wait()
        if jj + 1 < nb:
          pltpu.async_copy(msg.at[sv.at[jj + 1]], rows_bufs[(jj + 1) % 2],
                           sems[(jj + 1) % 2])

        def scale_g(g, _):
          wvec = wv[jj, pl.ds(g * 16, 16)]
          for lane in range(16):
            w_s = wvec[lane]
            k = g * 16 + lane
            for d8 in range(8):
              sl = pl.ds(d8 * 16, 16)
              cur[k, sl] = cur[k, sl] * w_s
          return 0
        lax.fori_loop(0, CHUNK // 16, scale_g, 0)
        pltpu.sync_copy(cur, acc.at[dv.at[jj]], add=True)
      return 0
    lax.fori_loop(0, n_blocks, block_body, 0)

  def stage_gg(bb):
    off = pl.multiple_of(bb * NB, NB)
    pltpu.sync_copy(ggs.at[t, pl.ds(off, NB)], sv)
    pltpu.sync_copy(ggd.at[t, pl.ds(off, NB)], dv)
    pltpu.sync_copy(ggw.at[t, pl.ds(off, NB)], wv)

  def stage_gp(bb):
    pltpu.sync_copy(gps.at[t, bb], sv.at[pl.ds(0, NBP)])
    pltpu.sync_copy(gpd.at[t, bb], dv.at[pl.ds(0, NBP)])
    pltpu.sync_copy(gpw.at[t, bb], wv.at[pl.ds(0, NBP)])

  do_rel(CG // NB, NB, stage_gg, msg_gg, acc_g)
  do_rel(CPB, NBP, stage_gp, msg_gp, acc_p)
  plsc.subcore_barrier()

  pltpu.sync_copy(acc_g.at[pl.ds(base_g, 624)], tg_out.at[c, pl.ds(base_g, 624)])
  pltpu.sync_copy(acc_p.at[pl.ds(s * 64, 64)], tp_out.at[c, pl.ds(s * 64, 64)])
  @pl.when(s == 0)
  def _():
    pltpu.sync_copy(acc_g.at[pl.ds(9984, 16)], tg_out.at[c, pl.ds(9984, 16)])


def _seg_kernel():
  return pl.kernel(
    _seg_body,
    out_type=(
        jax.ShapeDtypeStruct((NC, N_G, D), jnp.float32),
        jax.ShapeDtypeStruct((NC, N_P_PAD, D), jnp.float32),
    ),
    mesh=plsc.VectorSubcoreMesh(
        core_axis_name="c", subcore_axis_name="s", num_cores=NC,
        num_subcores=NS),
    scratch_types=(
        pltpu.VMEM_SHARED((N_G_PAD, D), jnp.float32),
        pltpu.VMEM_SHARED((N_P_PAD, D), jnp.float32),
        pltpu.VMEM((NB, CHUNK), jnp.int32),
        pltpu.VMEM((NB, CHUNK), jnp.int32),
        pltpu.VMEM((NB, CHUNK), jnp.float32),
        pltpu.VMEM((CHUNK, D), jnp.float32),
        pltpu.VMEM((CHUNK, D), jnp.float32),
        pltpu.SemaphoreType.DMA,
        pltpu.SemaphoreType.DMA,
    ),
  )


def _mm2_body(x_ref, w1_ref, b1_ref, w2_ref, b2_ref, o1_ref, o2_ref):
  x = x_ref[...]
  o1_ref[...] = jnp.dot(x, w1_ref[...],
                        preferred_element_type=jnp.float32) + b1_ref[...]
  o2_ref[...] = jnp.dot(x, w2_ref[...],
                        preferred_element_type=jnp.float32) + b2_ref[...]


def _mm2(h, w1, b1, w2, b2):
  blk = 2000
  grid = N_G // blk
  return pl.pallas_call(
      _mm2_body,
      grid=(grid,),
      in_specs=[
          pl.BlockSpec((blk, D), lambda i: (i, 0)),
          pl.BlockSpec((D, D), lambda i: (0, 0)),
          pl.BlockSpec((1, D), lambda i: (0, 0)),
          pl.BlockSpec((D, D), lambda i: (0, 0)),
          pl.BlockSpec((1, D), lambda i: (0, 0)),
      ],
      out_specs=[
          pl.BlockSpec((blk, D), lambda i: (i, 0)),
          pl.BlockSpec((blk, D), lambda i: (i, 0)),
      ],
      out_shape=[
          jax.ShapeDtypeStruct((N_G, D), jnp.float32),
          jax.ShapeDtypeStruct((N_G, D), jnp.float32),
      ],
  )(h, w1, b1.reshape(1, D), w2, b2.reshape(1, D))


def _finish_math(t, h, wa, ba, g, b, sk):
  z = jnp.dot(t, wa, preferred_element_type=jnp.float32) + ba
  a = jax.nn.sigmoid(sk)
  u = z * a + h * (1.0 - a)
  m = jnp.mean(u, axis=-1, keepdims=True)
  v = jnp.mean((u - m) ** 2, axis=-1, keepdims=True)
  return (u - m) * lax.rsqrt(v + 1e-5) * g + b


def _fin_body(sk_ref, t_ref, h_ref, wa_ref, ba_ref, g_ref, b_ref, o_ref):
  t = t_ref[0] + t_ref[1]
  o_ref[...] = _finish_math(t, h_ref[...], wa_ref[...], ba_ref[...],
                            g_ref[...], b_ref[...], sk_ref[0])


def _fin(tpart, h, wa, ba, g, b, sk, n, blk):
  grid = n // blk
  return pl.pallas_call(
      _fin_body,
      grid=(grid,),
      in_specs=[
          pl.BlockSpec(memory_space=pltpu.SMEM),
          pl.BlockSpec((NC, blk, D), lambda i: (0, i, 0)),
          pl.BlockSpec((blk, D), lambda i: (i, 0)),
          pl.BlockSpec((D, D), lambda i: (0, 0)),
          pl.BlockSpec((1, D), lambda i: (0, 0)),
          pl.BlockSpec((1, D), lambda i: (0, 0)),
          pl.BlockSpec((1, D), lambda i: (0, 0)),
      ],
      out_specs=pl.BlockSpec((blk, D), lambda i: (i, 0)),
      out_shape=jax.ShapeDtypeStruct((n, D), jnp.float32),
  )(jnp.reshape(sk, (1,)), tpart, h, wa, ba.reshape(1, D), g.reshape(1, D),
    b.reshape(1, D))


def _fin_head_body(sk_ref, t_ref, h_ref, wa_ref, ba_ref, g_ref, b_ref,
                   wo_ref, bo_ref, o_ref):
  t = t_ref[0] + t_ref[1]
  hp = _finish_math(t, h_ref[...], wa_ref[...], ba_ref[...],
                    g_ref[...], b_ref[...], sk_ref[0])
  o_ref[...] = jnp.dot(hp, wo_ref[...],
                       preferred_element_type=jnp.float32) + bo_ref[...]


def _fin_head(tpart, h, wa, ba, g, b, sk, wo, bo):
  return pl.pallas_call(
      _fin_head_body,
      grid=(1,),
      in_specs=[
          pl.BlockSpec(memory_space=pltpu.SMEM),
          pl.BlockSpec((NC, N_P, D), lambda i: (0, 0, 0)),
          pl.BlockSpec((N_P, D), lambda i: (0, 0)),
          pl.BlockSpec((D, D), lambda i: (0, 0)),
          pl.BlockSpec((1, D), lambda i: (0, 0)),
          pl.BlockSpec((1, D), lambda i: (0, 0)),
          pl.BlockSpec((1, D), lambda i: (0, 0)),
          pl.BlockSpec((D, OUT), lambda i: (0, 0)),
          pl.BlockSpec((1, OUT), lambda i: (0, 0)),
      ],
      out_specs=pl.BlockSpec((N_P, OUT), lambda i: (0, 0)),
      out_shape=jax.ShapeDtypeStruct((N_P, OUT), jnp.float32),
  )(jnp.reshape(sk, (1,)), tpart, h, wa, ba.reshape(1, D), g.reshape(1, D),
    b.reshape(1, D), wo, bo.reshape(1, OUT))


def _pad_edges(src, dst, w, shape, pad_row_base, pad_row_span):
  e = src.shape[0]
  tot = int(np.prod(shape))
  pad = tot - e
  pad_dst = pad_row_base + (jnp.arange(pad, dtype=jnp.int32) % pad_row_span)
  src = jnp.pad(src, (0, pad)).reshape(shape)
  dst = jnp.concatenate([dst, pad_dst]).reshape(shape)
  w = jnp.pad(w, (0, pad)).reshape(shape)
  return src, dst, w


def kernel(nid_gene, nid_patient, gg_src, gg_dst, gp_src, gp_dst, w_gg, w_gp,
           emb_gene, emb_patient, Wr_gg, br_gg, Wr_gp, br_gp,
           Wa_g, ba_g, Wa_p, ba_p, ln_g_w, ln_g_b, ln_p_w, ln_p_b,
           skip, W_out, b_out):
  h_g = jnp.take(emb_gene, nid_gene, axis=0)
  h_p = jnp.take(emb_patient, nid_patient, axis=0)

  ggs, ggd, ggw = _pad_edges(gg_src, gg_dst, w_gg, (NC * NS, CG, CHUNK),
                             N_G, N_G_PAD - N_G)
  gps, gpd, gpw = _pad_edges(gp_src, gp_dst, w_gp, (NC * NS, CPB, NBP, CHUNK),
                             N_P, N_P_PAD - N_P)

  seg = _seg_kernel()

  for l in range(2):
    msg_gg, msg_gp = _mm2(h_g, Wr_gg[l], br_gg[l], Wr_gp[l], br_gp[l])
    tg_part, tp_part = seg(msg_gg, msg_gp, ggs, ggd, ggw, gps, gpd, gpw)
    tp_part = tp_part[:, :N_P]
    wa_g = Wa_g[l]
    wa_p = Wa_p[l]
    if l == 0:
      h_g = _fin(tg_part, h_g, wa_g, ba_g[l], ln_g_w[l], ln_g_b[l],
                 skip[l, 0], N_G, 2000)
      h_p = _fin(tp_part, h_p, wa_p, ba_p[l], ln_p_w[l], ln_p_b[l],
                 skip[l, 1], N_P, N_P)
    else:
      logits = _fin_head(tp_part, h_p, wa_p, ba_p[l], ln_p_w[l],
                         ln_p_b[l], skip[l, 1], W_out, b_out)
  return logits


# speedup vs baseline: 3.8641x; 1.0012x over previous
import numpy as np
import jax
import jax.numpy as jnp
from jax import lax
from jax.experimental import pallas as pl
from jax.experimental.pallas import tpu as pltpu
from jax.experimental.pallas import tpu_sc as plsc

N_G = 10000
N_G_PAD = 10048
N_P = 1000
N_P_PAD = 1024
D = 128
OUT = 16
NC = 2
NS = 16
CHUNK = 128
NB = 8
CG = 80
NBP = 5
CPB = 5


def _seg_body(msg_gg, msg_gp, ggs, ggd, ggw, gps, gpd, gpw,
              tg_out, tp_out,
              acc_g, acc_p, sv, dv, wv, rows0, rows1, sem0, sem1,
              ssem0, ssem1):
  c = lax.axis_index("c")
  s = lax.axis_index("s")
  t = c * NS + s
  rows_bufs = (rows0, rows1)
  sems = (sem0, sem1)
  ssems = (ssem0, ssem1)
  rf = rows0

  def zrow(i, _):
    for d8 in range(8):
      rf[i, pl.ds(d8 * 16, 16)] = jnp.zeros((16,), jnp.float32)
    return 0
  lax.fori_loop(0, CHUNK, zrow, 0)

  base_g = s * 624
  for i in range(4):
    pltpu.sync_copy(rf, acc_g.at[pl.ds(base_g + i * 128, 128)])
  pltpu.sync_copy(rf.at[pl.ds(0, 112)], acc_g.at[pl.ds(base_g + 512, 112)])
  pltpu.sync_copy(rf.at[pl.ds(0, 64)], acc_p.at[pl.ds(s * 64, 64)])
  @pl.when(s == 0)
  def _():
    pltpu.sync_copy(rf.at[pl.ds(0, 64)], acc_g.at[pl.ds(9984, 64)])
  plsc.subcore_barrier()

  def do_rel(n_blocks, nb, stage, msg, acc):
    def block_body(bb, _):
      stage(bb)
      pltpu.async_copy(msg.at[sv.at[0]], rows_bufs[0], sems[0])
      for jj in range(nb):
        b = jj % 2
        o = (jj + 1) % 2
        cur = rows_bufs[b]
        pltpu.make_async_copy(msg.at[sv.at[jj]], cur, sems[b]).wait()
        if jj + 1 < nb:
          if jj >= 1:
            pltpu.make_async_copy(rows_bufs[o], acc.at[dv.at[jj - 1]],
                                  ssems[o]).wait()
          pltpu.async_copy(msg.at[sv.at[jj + 1]], rows_bufs[o], sems[o])

        def scale_g(g, _):
          wvec = wv[jj, pl.ds(g * 16, 16)]
          for lane in range(16):
            w_s = wvec[lane]
            k = g * 16 + lane
            for d8 in range(8):
              sl = pl.ds(d8 * 16, 16)
              cur[k, sl] = cur[k, sl] * w_s
          return 0
        lax.fori_loop(0, CHUNK // 16, scale_g, 0)
        if jj + 1 < nb:
          pltpu.async_copy(cur, acc.at[dv.at[jj]], ssems[b], add=True)
        else:
          pltpu.make_async_copy(rows_bufs[o], acc.at[dv.at[jj - 1]],
                                ssems[o]).wait()
          pltpu.sync_copy(cur, acc.at[dv.at[jj]], add=True)
      return 0
    lax.fori_loop(0, n_blocks, block_body, 0)

  def stage_gg(bb):
    off = pl.multiple_of(bb * NB, NB)
    pltpu.sync_copy(ggs.at[t, pl.ds(off, NB)], sv)
    pltpu.sync_copy(ggd.at[t, pl.ds(off, NB)], dv)
    pltpu.sync_copy(ggw.at[t, pl.ds(off, NB)], wv)

  def stage_gp(bb):
    pltpu.sync_copy(gps.at[t, bb], sv.at[pl.ds(0, NBP)])
    pltpu.sync_copy(gpd.at[t, bb], dv.at[pl.ds(0, NBP)])
    pltpu.sync_copy(gpw.at[t, bb], wv.at[pl.ds(0, NBP)])

  do_rel(CG // NB, NB, stage_gg, msg_gg, acc_g)
  do_rel(CPB, NBP, stage_gp, msg_gp, acc_p)
  plsc.subcore_barrier()

  pltpu.sync_copy(acc_g.at[pl.ds(base_g, 624)], tg_out.at[c, pl.ds(base_g, 624)])
  pltpu.sync_copy(acc_p.at[pl.ds(s * 64, 64)], tp_out.at[c, pl.ds(s * 64, 64)])
  @pl.when(s == 0)
  def _():
    pltpu.sync_copy(acc_g.at[pl.ds(9984, 16)], tg_out.at[c, pl.ds(9984, 16)])


def _seg_kernel():
  return pl.kernel(
    _seg_body,
    out_type=(
        jax.ShapeDtypeStruct((NC, N_G, D), jnp.float32),
        jax.ShapeDtypeStruct((NC, N_P_PAD, D), jnp.float32),
    ),
    mesh=plsc.VectorSubcoreMesh(
        core_axis_name="c", subcore_axis_name="s", num_cores=NC,
        num_subcores=NS),
    scratch_types=(
        pltpu.VMEM_SHARED((N_G_PAD, D), jnp.float32),
        pltpu.VMEM_SHARED((N_P_PAD, D), jnp.float32),
        pltpu.VMEM((NB, CHUNK), jnp.int32),
        pltpu.VMEM((NB, CHUNK), jnp.int32),
        pltpu.VMEM((NB, CHUNK), jnp.float32),
        pltpu.VMEM((CHUNK, D), jnp.float32),
        pltpu.VMEM((CHUNK, D), jnp.float32),
        pltpu.SemaphoreType.DMA,
        pltpu.SemaphoreType.DMA,
        pltpu.SemaphoreType.DMA,
        pltpu.SemaphoreType.DMA,
    ),
  )


def _mm2_body(x_ref, w1_ref, b1_ref, w2_ref, b2_ref, o1_ref, o2_ref):
  x = x_ref[...]
  o1_ref[...] = jnp.dot(x, w1_ref[...],
                        preferred_element_type=jnp.float32) + b1_ref[...]
  o2_ref[...] = jnp.dot(x, w2_ref[...],
                        preferred_element_type=jnp.float32) + b2_ref[...]


def _mm2(h, w1, b1, w2, b2):
  blk = 2000
  grid = N_G // blk
  return pl.pallas_call(
      _mm2_body,
      grid=(grid,),
      in_specs=[
          pl.BlockSpec((blk, D), lambda i: (i, 0)),
          pl.BlockSpec((D, D), lambda i: (0, 0)),
          pl.BlockSpec((1, D), lambda i: (0, 0)),
          pl.BlockSpec((D, D), lambda i: (0, 0)),
          pl.BlockSpec((1, D), lambda i: (0, 0)),
      ],
      out_specs=[
          pl.BlockSpec((blk, D), lambda i: (i, 0)),
          pl.BlockSpec((blk, D), lambda i: (i, 0)),
      ],
      out_shape=[
          jax.ShapeDtypeStruct((N_G, D), jnp.float32),
          jax.ShapeDtypeStruct((N_G, D), jnp.float32),
      ],
  )(h, w1, b1.reshape(1, D), w2, b2.reshape(1, D))


def _finish_math(t, h, wa, ba, g, b, sk):
  z = jnp.dot(t, wa, preferred_element_type=jnp.float32) + ba
  a = jax.nn.sigmoid(sk)
  u = z * a + h * (1.0 - a)
  m = jnp.mean(u, axis=-1, keepdims=True)
  v = jnp.mean((u - m) ** 2, axis=-1, keepdims=True)
  return (u - m) * lax.rsqrt(v + 1e-5) * g + b


def _fin_body(sk_ref, t_ref, h_ref, wa_ref, ba_ref, g_ref, b_ref, o_ref):
  t = t_ref[0] + t_ref[1]
  o_ref[...] = _finish_math(t, h_ref[...], wa_ref[...], ba_ref[...],
                            g_ref[...], b_ref[...], sk_ref[0])


def _fin(tpart, h, wa, ba, g, b, sk, n, blk):
  grid = n // blk
  return pl.pallas_call(
      _fin_body,
      grid=(grid,),
      in_specs=[
          pl.BlockSpec(memory_space=pltpu.SMEM),
          pl.BlockSpec((NC, blk, D), lambda i: (0, i, 0)),
          pl.BlockSpec((blk, D), lambda i: (i, 0)),
          pl.BlockSpec((D, D), lambda i: (0, 0)),
          pl.BlockSpec((1, D), lambda i: (0, 0)),
          pl.BlockSpec((1, D), lambda i: (0, 0)),
          pl.BlockSpec((1, D), lambda i: (0, 0)),
      ],
      out_specs=pl.BlockSpec((blk, D), lambda i: (i, 0)),
      out_shape=jax.ShapeDtypeStruct((n, D), jnp.float32),
  )(jnp.reshape(sk, (1,)), tpart, h, wa, ba.reshape(1, D), g.reshape(1, D),
    b.reshape(1, D))


def _fin_head_body(sk_ref, t_ref, h_ref, wa_ref, ba_ref, g_ref, b_ref,
                   wo_ref, bo_ref, o_ref):
  t = t_ref[0] + t_ref[1]
  hp = _finish_math(t, h_ref[...], wa_ref[...], ba_ref[...],
                    g_ref[...], b_ref[...], sk_ref[0])
  o_ref[...] = jnp.dot(hp, wo_ref[...],
                       preferred_element_type=jnp.float32) + bo_ref[...]


def _fin_head(tpart, h, wa, ba, g, b, sk, wo, bo):
  return pl.pallas_call(
      _fin_head_body,
      grid=(1,),
      in_specs=[
          pl.BlockSpec(memory_space=pltpu.SMEM),
          pl.BlockSpec((NC, N_P, D), lambda i: (0, 0, 0)),
          pl.BlockSpec((N_P, D), lambda i: (0, 0)),
          pl.BlockSpec((D, D), lambda i: (0, 0)),
          pl.BlockSpec((1, D), lambda i: (0, 0)),
          pl.BlockSpec((1, D), lambda i: (0, 0)),
          pl.BlockSpec((1, D), lambda i: (0, 0)),
          pl.BlockSpec((D, OUT), lambda i: (0, 0)),
          pl.BlockSpec((1, OUT), lambda i: (0, 0)),
      ],
      out_specs=pl.BlockSpec((N_P, OUT), lambda i: (0, 0)),
      out_shape=jax.ShapeDtypeStruct((N_P, OUT), jnp.float32),
  )(jnp.reshape(sk, (1,)), tpart, h, wa, ba.reshape(1, D), g.reshape(1, D),
    b.reshape(1, D), wo, bo.reshape(1, OUT))


def _pad_edges(src, dst, w, shape, pad_row_base, pad_row_span):
  e = src.shape[0]
  tot = int(np.prod(shape))
  pad = tot - e
  pad_dst = pad_row_base + (jnp.arange(pad, dtype=jnp.int32) % pad_row_span)
  src = jnp.pad(src, (0, pad)).reshape(shape)
  dst = jnp.concatenate([dst, pad_dst]).reshape(shape)
  w = jnp.pad(w, (0, pad)).reshape(shape)
  return src, dst, w


def kernel(nid_gene, nid_patient, gg_src, gg_dst, gp_src, gp_dst, w_gg, w_gp,
           emb_gene, emb_patient, Wr_gg, br_gg, Wr_gp, br_gp,
           Wa_g, ba_g, Wa_p, ba_p, ln_g_w, ln_g_b, ln_p_w, ln_p_b,
           skip, W_out, b_out):
  h_g = jnp.take(emb_gene, nid_gene, axis=0)
  h_p = jnp.take(emb_patient, nid_patient, axis=0)

  ggs, ggd, ggw = _pad_edges(gg_src, gg_dst, w_gg, (NC * NS, CG, CHUNK),
                             N_G, N_G_PAD - N_G)
  gps, gpd, gpw = _pad_edges(gp_src, gp_dst, w_gp, (NC * NS, CPB, NBP, CHUNK),
                             N_P, N_P_PAD - N_P)

  seg = _seg_kernel()

  for l in range(2):
    msg_gg, msg_gp = _mm2(h_g, Wr_gg[l], br_gg[l], Wr_gp[l], br_gp[l])
    tg_part, tp_part = seg(msg_gg, msg_gp, ggs, ggd, ggw, gps, gpd, gpw)
    tp_part = tp_part[:, :N_P]
    wa_g = Wa_g[l]
    wa_p = Wa_p[l]
    if l == 0:
      h_g = _fin(tg_part, h_g, wa_g, ba_g[l], ln_g_w[l], ln_g_b[l],
                 skip[l, 0], N_G, 2000)
      h_p = _fin(tp_part, h_p, wa_p, ba_p[l], ln_p_w[l], ln_p_b[l],
                 skip[l, 1], N_P, N_P)
    else:
      logits = _fin_head(tp_part, h_p, wa_p, ba_p[l], ln_p_w[l],
                         ln_p_b[l], skip[l, 1], W_out, b_out)
  return logits
